# Initial kernel scaffold; baseline (speedup 1.0000x reference)
import jax
import jax.numpy as jnp
from jax.experimental import pallas as pl


def kernel(x, token_table, pos_table):
    raise NotImplementedError("write your pallas kernel here")


# speedup vs baseline: 3.3463x; 3.3463x over previous
import functools

import jax
import jax.numpy as jnp
from jax import lax
from jax.experimental import pallas as pl
from jax.experimental.pallas import tpu as pltpu
from jax.experimental.pallas import tpu_sc as plsc

_NW = 32
_C = 400
_IW = 80
_LANES = 16


def _emb_body(idx_hbm, pos_hbm, tok_hbm, out_hbm, idx_v, rows_v, pos_v, sem,
              *, rows_per_w, seq_len, embed):
    nc = 2
    wid = lax.axis_index("s") * nc + lax.axis_index("c")
    base = wid * rows_per_w
    n_chunks = rows_per_w // _C
    n_gathers = _C // _IW
    vregs_per_row = embed // _LANES

    pltpu.sync_copy(pos_hbm, pos_v.at[pl.ds(0, seq_len), :])
    pltpu.sync_copy(pos_hbm, pos_v.at[pl.ds(seq_len, seq_len), :])

    def chunk_body(g, carry):
        rbase = base + g * _C
        pltpu.sync_copy(idx_hbm.at[pl.ds(rbase, _C)], idx_v)
        copies = []
        for j in range(n_gathers):
            copies.append(
                pltpu.async_copy(
                    tok_hbm.at[idx_v.at[pl.ds(j * _IW, _IW)]],
                    rows_v.at[pl.ds(j * _IW, _IW), :],
                    sem,
                )
            )
        for cp in copies:
            cp.wait()

        def add_body(r, c2):
            for u in range(vregs_per_row):
                sl = pl.ds(u * _LANES, _LANES)
                rows_v[r, sl] = rows_v[r, sl] + pos_v[r, sl]
            return c2

        lax.fori_loop(0, _C, add_body, 0)
        pltpu.sync_copy(rows_v, out_hbm.at[pl.ds(rbase, _C), :])
        return carry

    lax.fori_loop(0, n_chunks, chunk_body, 0)


def kernel(x, token_table, pos_table):
    batch, seq_len = x.shape
    _, embed = token_table.shape
    n = batch * seq_len
    rows_per_w = n // _NW

    idx_flat = x.reshape(n).astype(jnp.int32)

    mesh = plsc.VectorSubcoreMesh(core_axis_name="c", subcore_axis_name="s")
    body = functools.partial(
        _emb_body, rows_per_w=rows_per_w, seq_len=seq_len, embed=embed
    )
    out = pl.kernel(
        body,
        out_type=jax.ShapeDtypeStruct((n, embed), jnp.float32),
        mesh=mesh,
        scratch_types=[
            pltpu.VMEM((_C,), jnp.int32),
            pltpu.VMEM((_C, embed), jnp.float32),
            pltpu.VMEM((_C, embed), jnp.float32),
            pltpu.SemaphoreType.DMA,
        ],
        compiler_params=pltpu.CompilerParams(use_tc_tiling_on_sc=False),
    )(idx_flat, pos_table, token_table)
    return out.reshape(batch, seq_len, embed)


# speedup vs baseline: 3.9204x; 1.1715x over previous
import functools

import jax
import jax.numpy as jnp
from jax import lax
from jax.experimental import pallas as pl
from jax.experimental.pallas import tpu as pltpu
from jax.experimental.pallas import tpu_sc as plsc

_NW = 32
_C = 200
_NBUF = 4
_SPLITS = ((0, 128), (128, 72))
_LANES = 16


def _emb_body(idx_hbm, pos_hbm, tok_hbm, out_hbm, idx_v, rows_v, pos_v,
              sem_g, sem_s, *, rows_per_w, embed):
    nc = 2
    wid = lax.axis_index("s") * nc + lax.axis_index("c")
    base = wid * rows_per_w
    n_chunks = rows_per_w // _C
    vregs_per_row = embed // _LANES

    pltpu.sync_copy(pos_hbm, pos_v)

    def gather_copy(off, sz, b):
        return pltpu.make_async_copy(
            tok_hbm.at[idx_v.at[b, pl.ds(off, sz)]],
            rows_v.at[b, pl.ds(off, sz), :],
            sem_g.at[b],
        )

    def store_copy(rbase, b):
        return pltpu.make_async_copy(
            rows_v.at[b],
            out_hbm.at[pl.ds(rbase, _C), :],
            sem_s.at[b],
        )

    def fire(g, b, first):
        rbase = base + g * _C
        if not first:
            store_copy(rbase - _NBUF * _C, b).wait()
        pltpu.sync_copy(idx_hbm.at[pl.ds(rbase, _C)], idx_v.at[b])
        for off, sz in _SPLITS:
            gather_copy(off, sz, b).start()

    def proc(g, b):
        for off, sz in _SPLITS:
            gather_copy(off, sz, b).wait()

        @plsc.parallel_loop(0, _C, 1, unroll=4)
        def _(r):
            for u in range(vregs_per_row):
                sl = pl.ds(u * _LANES, _LANES)
                rows_v[b, r, sl] = rows_v[b, r, sl] + pos_v[r, sl]

        store_copy(base + g * _C, b).start()

    for b in range(_NBUF):
        fire(b, b, first=True)

    def loop_body(it, carry):
        g0 = it * _NBUF
        for b in range(_NBUF):
            proc(g0 + b, b)
        for b in range(_NBUF):
            fire(g0 + _NBUF + b, b, first=False)
        return carry

    lax.fori_loop(0, n_chunks // _NBUF - 1, loop_body, 0)

    g_last = n_chunks - _NBUF
    for b in range(_NBUF):
        proc(g_last + b, b)
    for b in range(_NBUF):
        store_copy(base + (g_last + b) * _C, b).wait()


def kernel(x, token_table, pos_table):
    batch, seq_len = x.shape
    _, embed = token_table.shape
    n = batch * seq_len
    rows_per_w = n // _NW

    idx_flat = x.reshape(n).astype(jnp.int32)

    mesh = plsc.VectorSubcoreMesh(core_axis_name="c", subcore_axis_name="s")
    body = functools.partial(_emb_body, rows_per_w=rows_per_w, embed=embed)
    out = pl.kernel(
        body,
        out_type=jax.ShapeDtypeStruct((n, embed), jnp.float32),
        mesh=mesh,
        scratch_types=[
            pltpu.VMEM((_NBUF, _C), jnp.int32),
            pltpu.VMEM((_NBUF, _C, embed), jnp.float32),
            pltpu.VMEM((_C, embed), jnp.float32),
            pltpu.SemaphoreType.DMA((_NBUF,)),
            pltpu.SemaphoreType.DMA((_NBUF,)),
        ],
        compiler_params=pltpu.CompilerParams(use_tc_tiling_on_sc=False),
    )(idx_flat, pos_table, token_table)
    return out.reshape(batch, seq_len, embed)
